# Initial kernel scaffold; baseline (speedup 1.0000x reference)
import jax
import jax.numpy as jnp
from jax.experimental import pallas as pl


def kernel(x, pos_embedding):
    raise NotImplementedError("write your pallas kernel here")


# speedup vs baseline: 2.5102x; 2.5102x over previous
import jax
import jax.numpy as jnp
from jax.experimental import pallas as pl


def _add_body(x_ref, pe_ref, o_ref):
    o_ref[...] = x_ref[...] + pe_ref[...]


def kernel(x, pos_embedding):
    B, S, D = x.shape
    TS = 512
    return pl.pallas_call(
        _add_body,
        grid=(S // TS, B),
        in_specs=[
            pl.BlockSpec((1, TS, D), lambda s, b: (b, s, 0)),
            pl.BlockSpec((TS, D), lambda s, b: (s, 0)),
        ],
        out_specs=pl.BlockSpec((1, TS, D), lambda s, b: (b, s, 0)),
        out_shape=jax.ShapeDtypeStruct(x.shape, x.dtype),
    )(x, pos_embedding)


# speedup vs baseline: 2.6131x; 1.0410x over previous
import jax
import jax.numpy as jnp
from jax.experimental import pallas as pl


def _add_body(x_ref, pe_ref, o_ref):
    o_ref[...] = x_ref[...] + pe_ref[...]


def kernel(x, pos_embedding):
    B, S, D = x.shape
    TS = 1024
    return pl.pallas_call(
        _add_body,
        grid=(S // TS, B),
        in_specs=[
            pl.BlockSpec((1, TS, D), lambda s, b: (b, s, 0)),
            pl.BlockSpec((TS, D), lambda s, b: (s, 0)),
        ],
        out_specs=pl.BlockSpec((1, TS, D), lambda s, b: (b, s, 0)),
        out_shape=jax.ShapeDtypeStruct(x.shape, x.dtype),
    )(x, pos_embedding)
